# baseline (speedup 1.0000x reference)
import jax
import jax.numpy as jnp
from jax import lax
from jax.experimental import pallas as pl
from jax.experimental.pallas import tpu as pltpu
from jax.experimental.pallas import tpu_sc as plsc

N = 10000
NPAD = 10240
D_IN = 128
HID = 64
W2H = 128
E = 320000
E_TOT = E + N
NW = 32
CH = 64
IB = 18
NBLK = 9
NCH = IB * NBLK
EPW = NCH * CH
E_PAD = EPW * NW
ROWS_PT = NPAD // 16
BLK = 1024

f32 = jnp.float32
i32 = jnp.int32


def _dense1_body(x_ref, w_ref, a_ref, hs_ref, ad_ref, mx_ref):
    i = pl.program_id(0)
    h = jnp.dot(x_ref[...], w_ref[...], preferred_element_type=f32)
    a = jnp.dot(h, a_ref[...], preferred_element_type=f32)
    hs_ref[...] = jnp.concatenate([h, a[:, :HID]], axis=1)
    ad_ref[...] = a[:, HID:]
    bmax = jnp.broadcast_to(jnp.max(a, axis=0, keepdims=True), (8, 2 * HID))

    @pl.when(i == 0)
    def _():
        mx_ref[...] = bmax

    @pl.when(i > 0)
    def _():
        mx_ref[...] = jnp.maximum(mx_ref[...], bmax)


def _dense1(xp, W1, A1):
    grid = NPAD // BLK
    return pl.pallas_call(
        _dense1_body,
        grid=(grid,),
        in_specs=[
            pl.BlockSpec((BLK, D_IN), lambda i: (i, 0)),
            pl.BlockSpec((D_IN, HID), lambda i: (0, 0)),
            pl.BlockSpec((HID, 2 * HID), lambda i: (0, 0)),
        ],
        out_specs=[
            pl.BlockSpec((BLK, W2H), lambda i: (i, 0)),
            pl.BlockSpec((BLK, HID), lambda i: (i, 0)),
            pl.BlockSpec((8, 2 * HID), lambda i: (0, 0)),
        ],
        out_shape=[
            jax.ShapeDtypeStruct((NPAD, W2H), f32),
            jax.ShapeDtypeStruct((NPAD, HID), f32),
            jax.ShapeDtypeStruct((8, 2 * HID), f32),
        ],
    )(xp, W1, A1)


def _elu(x):
    return jnp.where(x > 0, x, jnp.exp(jnp.minimum(x, 0.0)) - 1.0)


def _dense2_body(nd0_ref, nd1_ref, b1_ref, w_ref, a_ref,
                 hs_ref, ad_ref, mx_ref):
    i = pl.program_id(0)
    nd = nd0_ref[...] + nd1_ref[...]
    num = nd[:, :HID]
    den = nd[:, HID:]
    g = _elu(num / (den + 1e-16) + b1_ref[...])
    h = jnp.dot(g, w_ref[...], preferred_element_type=f32)
    a = jnp.dot(h, a_ref[...], preferred_element_type=f32)
    hs_ref[...] = jnp.concatenate([h, a[:, :HID]], axis=1)
    ad_ref[...] = a[:, HID:]
    bmax = jnp.broadcast_to(jnp.max(a, axis=0, keepdims=True), (8, 2 * HID))

    @pl.when(i == 0)
    def _():
        mx_ref[...] = bmax

    @pl.when(i > 0)
    def _():
        mx_ref[...] = jnp.maximum(mx_ref[...], bmax)


def _dense2(nd0, nd1, b1r, W2, A2):
    grid = NPAD // BLK
    return pl.pallas_call(
        _dense2_body,
        grid=(grid,),
        in_specs=[
            pl.BlockSpec((BLK, W2H), lambda i: (i, 0)),
            pl.BlockSpec((BLK, W2H), lambda i: (i, 0)),
            pl.BlockSpec((1, HID), lambda i: (0, 0)),
            pl.BlockSpec((HID, HID), lambda i: (0, 0)),
            pl.BlockSpec((HID, 2 * HID), lambda i: (0, 0)),
        ],
        out_specs=[
            pl.BlockSpec((BLK, W2H), lambda i: (i, 0)),
            pl.BlockSpec((BLK, HID), lambda i: (i, 0)),
            pl.BlockSpec((8, 2 * HID), lambda i: (0, 0)),
        ],
        out_shape=[
            jax.ShapeDtypeStruct((NPAD, W2H), f32),
            jax.ShapeDtypeStruct((NPAD, HID), f32),
            jax.ShapeDtypeStruct((8, 2 * HID), f32),
        ],
    )(nd0, nd1, b1r, W2, A2)


def _final_body(nd0_ref, nd1_ref, b2_ref, o_ref):
    nd = nd0_ref[...] + nd1_ref[...]
    o_ref[...] = nd[:, :HID] / (nd[:, HID:] + 1e-16) + b2_ref[...]


def _final(nd0, nd1, b2r):
    grid = NPAD // BLK
    return pl.pallas_call(
        _final_body,
        grid=(grid,),
        in_specs=[
            pl.BlockSpec((BLK, W2H), lambda i: (i, 0)),
            pl.BlockSpec((BLK, W2H), lambda i: (i, 0)),
            pl.BlockSpec((1, HID), lambda i: (0, 0)),
        ],
        out_specs=pl.BlockSpec((BLK, HID), lambda i: (i, 0)),
        out_shape=jax.ShapeDtypeStruct((NPAD, HID), f32),
    )(nd0, nd1, b2r)


def _edge_body(src_hbm, dst_hbm, hs_hbm, ad_hbm, m_hbm,
               nd_hbm,
               srcb, dstb, mb,
               hsv0, adv0, hw0,
               hsv1, adv1, hw1,
               zb, acc_sh,
               gsem0, gsem1, ssem0, ssem1):
    c = lax.axis_index("c")
    s = lax.axis_index("s")
    wid = c * 16 + s
    row0 = s * ROWS_PT

    zero = jnp.zeros((16,), f32)

    def zrow(i, _):
        for j in range(8):
            zb[i, pl.ds(16 * j, 16)] = zero
        return 0

    lax.fori_loop(0, 32, zrow, 0)
    for r in range(ROWS_PT // 32):
        pltpu.sync_copy(zb, acc_sh.at[pl.ds(row0 + 32 * r, 32)])
    plsc.subcore_barrier()

    pltpu.sync_copy(m_hbm, mb)
    M = [mb[pl.ds(16 * j, 16)] for j in range(4)]

    slots = [
        (hsv0, adv0, hw0, gsem0, ssem0),
        (hsv1, adv1, hw1, gsem1, ssem1),
    ]

    def issue_gathers(ch, b):
        hsv, adv, _, gsem, _ = slots[b]
        pltpu.async_copy(hs_hbm.at[srcb.at[ch]], hsv, gsem)
        pltpu.async_copy(ad_hbm.at[dstb.at[ch]], adv, gsem)

    def wait_gathers(b):
        hsv, adv, _, gsem, _ = slots[b]
        pltpu.make_async_copy(hs_hbm.at[srcb.at[0]], hsv, gsem).wait()
        pltpu.make_async_copy(ad_hbm.at[dstb.at[0]], adv, gsem).wait()

    def issue_scatter(ch, b):
        _, _, hw, _, ssem = slots[b]
        pltpu.async_copy(hw, acc_sh.at[dstb.at[ch]], ssem, add=True)

    def wait_scatter(b):
        _, _, hw, _, ssem = slots[b]
        pltpu.make_async_copy(hw, acc_sh.at[dstb.at[0]], ssem).wait()

    def compute(b):
        hsv, adv, hw, _, _ = slots[b]

        @plsc.parallel_loop(0, CH, 1, unroll=4)
        def _(i):
            for j in range(4):
                sl = pl.ds(16 * j, 16)
                sh = pl.ds(HID + 16 * j, 16)
                u = hsv[i, sh] + adv[i, sl]
                e = jnp.where(u >= 0, u, 0.2 * u)
                p = jnp.exp(e - M[j])
                hw[i, sh] = p
                hw[i, sl] = hsv[i, sl] * p

    def block(blk, _):
        pltpu.sync_copy(src_hbm.at[wid, pl.ds(blk * IB, IB)], srcb)
        pltpu.sync_copy(dst_hbm.at[wid, pl.ds(blk * IB, IB)], dstb)
        issue_gathers(0, 0)

        def pair(k, _):
            for b in (0, 1):
                ch = 2 * k + b
                wait_gathers(b)
                if b == 0:
                    issue_gathers(ch + 1, 1)
                else:
                    @pl.when(k < IB // 2 - 1)
                    def _():
                        issue_gathers(ch + 1, 0)

                @pl.when(k >= 1)
                def _():
                    wait_scatter(b)
                compute(b)
                issue_scatter(ch, b)
            return 0

        lax.fori_loop(0, IB // 2, pair, 0)
        wait_scatter(0)
        wait_scatter(1)
        return 0

    lax.fori_loop(0, NBLK, block, 0)
    plsc.subcore_barrier()

    pltpu.sync_copy(acc_sh.at[pl.ds(row0, ROWS_PT)],
                    nd_hbm.at[c, pl.ds(row0, ROWS_PT)])


_edge = pl.kernel(
    _edge_body,
    out_type=jax.ShapeDtypeStruct((2, NPAD, W2H), f32),
    mesh=plsc.VectorSubcoreMesh(core_axis_name="c", subcore_axis_name="s",
                                num_cores=2, num_subcores=16),
    scratch_types=[
        pltpu.VMEM((IB, CH), i32),
        pltpu.VMEM((IB, CH), i32),
        pltpu.VMEM((HID,), f32),
        pltpu.VMEM((CH, W2H), f32),
        pltpu.VMEM((CH, HID), f32),
        pltpu.VMEM((CH, W2H), f32),
        pltpu.VMEM((CH, W2H), f32),
        pltpu.VMEM((CH, HID), f32),
        pltpu.VMEM((CH, W2H), f32),
        pltpu.VMEM((32, W2H), f32),
        pltpu.VMEM_SHARED((NPAD, W2H), f32),
        pltpu.SemaphoreType.DMA,
        pltpu.SemaphoreType.DMA,
        pltpu.SemaphoreType.DMA,
        pltpu.SemaphoreType.DMA,
    ],
    compiler_params=pltpu.CompilerParams(use_tc_tiling_on_sc=False),
)


def _lrelu(x):
    return jnp.where(x >= 0, x, 0.2 * x)


def kernel(x, edge_index, W1, a1s, a1d, b1, W2, a2s, a2d, b2):
    loops = jnp.arange(N, dtype=i32)
    src = jnp.concatenate([
        edge_index[0].astype(i32), loops,
        jnp.zeros((E_PAD - E_TOT,), i32)])
    dst = jnp.concatenate([
        edge_index[1].astype(i32), loops,
        jnp.full((E_PAD - E_TOT,), N, i32)])
    srcg = src.reshape(NW, NCH, CH)
    dstg = dst.reshape(NW, NCH, CH)

    xp = jnp.pad(x, ((0, NPAD - N), (0, 0)))

    eye8 = jnp.eye(8, dtype=f32)
    ones8 = jnp.ones((1, 1, 1, 8), f32)
    A1s = (a1s[0][:, :, None, None] * eye8[:, None, :, None] * ones8
           ).reshape(HID, HID)
    A1d = (a1d[0][:, :, None, None] * eye8[:, None, :, None] * ones8
           ).reshape(HID, HID)
    A1 = jnp.concatenate([A1s, A1d], axis=1)
    A2s = a2s[0, 0][:, None] * jnp.ones((1, HID), f32)
    A2d = a2d[0, 0][:, None] * jnp.ones((1, HID), f32)
    A2 = jnp.concatenate([A2s, A2d], axis=1)
    b1r = b1.reshape(1, HID)
    b2r = b2.reshape(1, HID)

    hs1, ad1, mx1 = _dense1(xp, W1, A1)
    m1 = _lrelu(mx1[0, :HID] + mx1[0, HID:])
    nd1 = _edge(srcg, dstg, hs1, ad1, m1)

    hs2, ad2, mx2 = _dense2(nd1[0], nd1[1], b1r, W2, A2)
    m2 = _lrelu(mx2[0, :HID] + mx2[0, HID:])
    nd2 = _edge(srcg, dstg, hs2, ad2, m2)

    out = _final(nd2[0], nd2[1], b2r)
    return out[:N]


# speedup vs baseline: 1.0418x; 1.0418x over previous
import jax
import jax.numpy as jnp
from jax import lax
from jax.experimental import pallas as pl
from jax.experimental.pallas import tpu as pltpu
from jax.experimental.pallas import tpu_sc as plsc

N = 10000
NPAD = 10240
D_IN = 128
HID = 64
E = 320000
E_TOT = E + N
NW = 32
CH = 64
IB = 18
NBLK = 9
NCH = IB * NBLK
EPW = NCH * CH
E_PAD = EPW * NW
ROWS_PT = NPAD // 16
BLK = 1024

f32 = jnp.float32
i32 = jnp.int32


def _dense1_body(x_ref, w_ref, a_ref, h_ref, as_ref, ad_ref, mx_ref):
    i = pl.program_id(0)
    h = jnp.dot(x_ref[...], w_ref[...], preferred_element_type=f32)
    h_ref[...] = h
    a = jnp.dot(h, a_ref[...], preferred_element_type=f32)
    as_ref[...] = a[:, :HID]
    ad_ref[...] = a[:, HID:]
    bmax = jnp.broadcast_to(jnp.max(a, axis=0, keepdims=True), (8, 2 * HID))

    @pl.when(i == 0)
    def _():
        mx_ref[...] = bmax

    @pl.when(i > 0)
    def _():
        mx_ref[...] = jnp.maximum(mx_ref[...], bmax)


def _dense1(xp, W1, A1):
    grid = NPAD // BLK
    return pl.pallas_call(
        _dense1_body,
        grid=(grid,),
        in_specs=[
            pl.BlockSpec((BLK, D_IN), lambda i: (i, 0)),
            pl.BlockSpec((D_IN, HID), lambda i: (0, 0)),
            pl.BlockSpec((HID, 2 * HID), lambda i: (0, 0)),
        ],
        out_specs=[
            pl.BlockSpec((BLK, HID), lambda i: (i, 0)),
            pl.BlockSpec((BLK, HID), lambda i: (i, 0)),
            pl.BlockSpec((BLK, HID), lambda i: (i, 0)),
            pl.BlockSpec((8, 2 * HID), lambda i: (0, 0)),
        ],
        out_shape=[
            jax.ShapeDtypeStruct((NPAD, HID), f32),
            jax.ShapeDtypeStruct((NPAD, HID), f32),
            jax.ShapeDtypeStruct((NPAD, HID), f32),
            jax.ShapeDtypeStruct((8, 2 * HID), f32),
        ],
    )(xp, W1, A1)


def _elu(x):
    return jnp.where(x > 0, x, jnp.exp(jnp.minimum(x, 0.0)) - 1.0)


def _dense2_body(n0_ref, n1_ref, d0_ref, d1_ref, b1_ref, w_ref, a_ref,
                 h_ref, as_ref, ad_ref, mx_ref):
    i = pl.program_id(0)
    num = n0_ref[...] + n1_ref[...]
    den = d0_ref[...] + d1_ref[...]
    g = _elu(num / (den + 1e-16) + b1_ref[...])
    h = jnp.dot(g, w_ref[...], preferred_element_type=f32)
    h_ref[...] = h
    a = jnp.dot(h, a_ref[...], preferred_element_type=f32)
    as_ref[...] = a[:, :HID]
    ad_ref[...] = a[:, HID:]
    bmax = jnp.broadcast_to(jnp.max(a, axis=0, keepdims=True), (8, 2 * HID))

    @pl.when(i == 0)
    def _():
        mx_ref[...] = bmax

    @pl.when(i > 0)
    def _():
        mx_ref[...] = jnp.maximum(mx_ref[...], bmax)


def _dense2(n0, n1, d0, d1, b1r, W2, A2):
    grid = NPAD // BLK
    return pl.pallas_call(
        _dense2_body,
        grid=(grid,),
        in_specs=[
            pl.BlockSpec((BLK, HID), lambda i: (i, 0)),
            pl.BlockSpec((BLK, HID), lambda i: (i, 0)),
            pl.BlockSpec((BLK, HID), lambda i: (i, 0)),
            pl.BlockSpec((BLK, HID), lambda i: (i, 0)),
            pl.BlockSpec((1, HID), lambda i: (0, 0)),
            pl.BlockSpec((HID, HID), lambda i: (0, 0)),
            pl.BlockSpec((HID, 2 * HID), lambda i: (0, 0)),
        ],
        out_specs=[
            pl.BlockSpec((BLK, HID), lambda i: (i, 0)),
            pl.BlockSpec((BLK, HID), lambda i: (i, 0)),
            pl.BlockSpec((BLK, HID), lambda i: (i, 0)),
            pl.BlockSpec((8, 2 * HID), lambda i: (0, 0)),
        ],
        out_shape=[
            jax.ShapeDtypeStruct((NPAD, HID), f32),
            jax.ShapeDtypeStruct((NPAD, HID), f32),
            jax.ShapeDtypeStruct((NPAD, HID), f32),
            jax.ShapeDtypeStruct((8, 2 * HID), f32),
        ],
    )(n0, n1, d0, d1, b1r, W2, A2)


def _final_body(n0_ref, n1_ref, d0_ref, d1_ref, b2_ref, o_ref):
    num = n0_ref[...] + n1_ref[...]
    den = d0_ref[...] + d1_ref[...]
    o_ref[...] = num / (den + 1e-16) + b2_ref[...]


def _final(n0, n1, d0, d1, b2r):
    grid = NPAD // BLK
    return pl.pallas_call(
        _final_body,
        grid=(grid,),
        in_specs=[
            pl.BlockSpec((BLK, HID), lambda i: (i, 0)),
            pl.BlockSpec((BLK, HID), lambda i: (i, 0)),
            pl.BlockSpec((BLK, HID), lambda i: (i, 0)),
            pl.BlockSpec((BLK, HID), lambda i: (i, 0)),
            pl.BlockSpec((1, HID), lambda i: (0, 0)),
        ],
        out_specs=pl.BlockSpec((BLK, HID), lambda i: (i, 0)),
        out_shape=jax.ShapeDtypeStruct((N, HID), f32),
    )(n0, n1, d0, d1, b2r)


def _edge_body(src_hbm, dst_hbm, h_hbm, as_hbm, ad_hbm, m_hbm,
               num_hbm, den_hbm,
               srcb, dstb, mb,
               asv0, adv0, hg0, hw0, pv0,
               asv1, adv1, hg1, hw1, pv1,
               z64, num_sh, den_sh,
               gsem0, gsem1, ssem0, ssem1):
    c = lax.axis_index("c")
    s = lax.axis_index("s")
    wid = c * 16 + s
    row0 = s * ROWS_PT

    zero = jnp.zeros((16,), f32)

    def zrow(i, _):
        for j in range(4):
            z64[i, pl.ds(16 * j, 16)] = zero
        return 0

    lax.fori_loop(0, 64, zrow, 0)
    for r in range(ROWS_PT // 64):
        pltpu.sync_copy(z64, num_sh.at[pl.ds(row0 + 64 * r, 64)])
        pltpu.sync_copy(z64, den_sh.at[pl.ds(row0 + 64 * r, 64)])
    plsc.subcore_barrier()

    pltpu.sync_copy(m_hbm, mb)
    M = [mb[pl.ds(16 * j, 16)] for j in range(4)]

    slots = [
        (asv0, adv0, hg0, hw0, pv0, gsem0, ssem0),
        (asv1, adv1, hg1, hw1, pv1, gsem1, ssem1),
    ]

    def issue_gathers(ch, b):
        asb, adb, hg, _, _, gsem, _ = slots[b]
        pltpu.async_copy(as_hbm.at[srcb.at[ch]], asb, gsem)
        pltpu.async_copy(ad_hbm.at[dstb.at[ch]], adb, gsem)
        pltpu.async_copy(h_hbm.at[srcb.at[ch]], hg, gsem)

    def wait_gathers(b):
        asb, adb, hg, _, _, gsem, _ = slots[b]
        pltpu.make_async_copy(as_hbm.at[srcb.at[0]], asb, gsem).wait()
        pltpu.make_async_copy(ad_hbm.at[dstb.at[0]], adb, gsem).wait()
        pltpu.make_async_copy(h_hbm.at[srcb.at[0]], hg, gsem).wait()

    def issue_scatters(ch, b):
        _, _, _, hw, pb, _, ssem = slots[b]
        pltpu.async_copy(hw, num_sh.at[dstb.at[ch]], ssem, add=True)
        pltpu.async_copy(pb, den_sh.at[dstb.at[ch]], ssem, add=True)

    def wait_scatters(b):
        _, _, _, hw, pb, _, ssem = slots[b]
        pltpu.make_async_copy(hw, num_sh.at[dstb.at[0]], ssem).wait()
        pltpu.make_async_copy(pb, den_sh.at[dstb.at[0]], ssem).wait()

    def compute(b):
        asb, adb, hg, hw, pb, _, _ = slots[b]

        @plsc.parallel_loop(0, CH, 1, unroll=4)
        def _(i):
            for j in range(4):
                sl = pl.ds(16 * j, 16)
                u = asb[i, sl] + adb[i, sl]
                e = jnp.where(u >= 0, u, 0.2 * u)
                p = jnp.exp(e - M[j])
                pb[i, sl] = p
                hw[i, sl] = hg[i, sl] * p

    def block(blk, _):
        pltpu.sync_copy(src_hbm.at[wid, pl.ds(blk * IB, IB)], srcb)
        pltpu.sync_copy(dst_hbm.at[wid, pl.ds(blk * IB, IB)], dstb)
        issue_gathers(0, 0)

        def pair(k, _):
            for b in (0, 1):
                ch = 2 * k + b
                wait_gathers(b)
                if b == 0:
                    issue_gathers(ch + 1, 1)
                else:
                    @pl.when(k < IB // 2 - 1)
                    def _():
                        issue_gathers(ch + 1, 0)

                @pl.when(k >= 1)
                def _():
                    wait_scatters(b)
                compute(b)
                issue_scatters(ch, b)
            return 0

        lax.fori_loop(0, IB // 2, pair, 0)
        wait_scatters(0)
        wait_scatters(1)
        return 0

    lax.fori_loop(0, NBLK, block, 0)
    plsc.subcore_barrier()

    pltpu.sync_copy(num_sh.at[pl.ds(row0, ROWS_PT)],
                    num_hbm.at[c, pl.ds(row0, ROWS_PT)])
    pltpu.sync_copy(den_sh.at[pl.ds(row0, ROWS_PT)],
                    den_hbm.at[c, pl.ds(row0, ROWS_PT)])


_edge = pl.kernel(
    _edge_body,
    out_type=(
        jax.ShapeDtypeStruct((2, NPAD, HID), f32),
        jax.ShapeDtypeStruct((2, NPAD, HID), f32),
    ),
    mesh=plsc.VectorSubcoreMesh(core_axis_name="c", subcore_axis_name="s",
                                num_cores=2, num_subcores=16),
    scratch_types=[
        pltpu.VMEM((IB, CH), i32),
        pltpu.VMEM((IB, CH), i32),
        pltpu.VMEM((HID,), f32),
        pltpu.VMEM((CH, HID), f32),
        pltpu.VMEM((CH, HID), f32),
        pltpu.VMEM((CH, HID), f32),
        pltpu.VMEM((CH, HID), f32),
        pltpu.VMEM((CH, HID), f32),
        pltpu.VMEM((CH, HID), f32),
        pltpu.VMEM((CH, HID), f32),
        pltpu.VMEM((CH, HID), f32),
        pltpu.VMEM((CH, HID), f32),
        pltpu.VMEM((CH, HID), f32),
        pltpu.VMEM((64, HID), f32),
        pltpu.VMEM_SHARED((NPAD, HID), f32),
        pltpu.VMEM_SHARED((NPAD, HID), f32),
        pltpu.SemaphoreType.DMA,
        pltpu.SemaphoreType.DMA,
        pltpu.SemaphoreType.DMA,
        pltpu.SemaphoreType.DMA,
    ],
    compiler_params=pltpu.CompilerParams(use_tc_tiling_on_sc=False),
)


def _lrelu(x):
    return jnp.where(x >= 0, x, 0.2 * x)


def kernel(x, edge_index, W1, a1s, a1d, b1, W2, a2s, a2d, b2):
    loops = jnp.arange(N, dtype=i32)
    src = jnp.concatenate([
        edge_index[0].astype(i32), loops,
        jnp.zeros((E_PAD - E_TOT,), i32)])
    dst = jnp.concatenate([
        edge_index[1].astype(i32), loops,
        jnp.full((E_PAD - E_TOT,), N, i32)])
    srcg = src.reshape(NW, NCH, CH)
    dstg = dst.reshape(NW, NCH, CH)

    xp = jnp.pad(x, ((0, NPAD - N), (0, 0)))

    eye8 = jnp.eye(8, dtype=f32)
    ones8 = jnp.ones((1, 1, 1, 8), f32)
    A1s = (a1s[0][:, :, None, None] * eye8[:, None, :, None] * ones8
           ).reshape(HID, HID)
    A1d = (a1d[0][:, :, None, None] * eye8[:, None, :, None] * ones8
           ).reshape(HID, HID)
    A1 = jnp.concatenate([A1s, A1d], axis=1)
    A2s = a2s[0, 0][:, None] * jnp.ones((1, HID), f32)
    A2d = a2d[0, 0][:, None] * jnp.ones((1, HID), f32)
    A2 = jnp.concatenate([A2s, A2d], axis=1)
    b1r = b1.reshape(1, HID)
    b2r = b2.reshape(1, HID)

    h1, as1, ad1, mx1 = _dense1(xp, W1, A1)
    m1 = _lrelu(mx1[0, :HID] + mx1[0, HID:])
    num1, den1 = _edge(srcg, dstg, h1, as1, ad1, m1)

    h2, as2, ad2, mx2 = _dense2(num1[0], num1[1], den1[0], den1[1],
                                b1r, W2, A2)
    m2 = _lrelu(mx2[0, :HID] + mx2[0, HID:])
    num2, den2 = _edge(srcg, dstg, h2, as2, ad2, m2)

    return _final(num2[0], num2[1], den2[0], den2[1], b2r)


# speedup vs baseline: 1.0481x; 1.0061x over previous
import jax
import jax.numpy as jnp
from jax import lax
from jax.experimental import pallas as pl
from jax.experimental.pallas import tpu as pltpu
from jax.experimental.pallas import tpu_sc as plsc

N = 10000
NPAD = 10240
D_IN = 128
HID = 64
E = 320000
E_TOT = E + N
NW = 32
CH = 64
IB = 18
NBLK = 9
NCH = IB * NBLK
EPW = NCH * CH
E_PAD = EPW * NW
ROWS_PT = NPAD // 16
BLK = 1024

f32 = jnp.float32
i32 = jnp.int32


def _dense1_body(x_ref, w_ref, a_ref, h_ref, as_ref, ad_ref, mx_ref):
    i = pl.program_id(0)
    h = jnp.dot(x_ref[...], w_ref[...], preferred_element_type=f32)
    h_ref[...] = h
    a = jnp.dot(h, a_ref[...], preferred_element_type=f32)
    as_ref[...] = a[:, :HID]
    ad_ref[...] = a[:, HID:]
    bmax = jnp.broadcast_to(jnp.max(a, axis=0, keepdims=True), (8, 2 * HID))

    @pl.when(i == 0)
    def _():
        mx_ref[...] = bmax

    @pl.when(i > 0)
    def _():
        mx_ref[...] = jnp.maximum(mx_ref[...], bmax)


def _dense1(xp, W1, A1):
    grid = NPAD // BLK
    return pl.pallas_call(
        _dense1_body,
        grid=(grid,),
        in_specs=[
            pl.BlockSpec((BLK, D_IN), lambda i: (i, 0)),
            pl.BlockSpec((D_IN, HID), lambda i: (0, 0)),
            pl.BlockSpec((HID, 2 * HID), lambda i: (0, 0)),
        ],
        out_specs=[
            pl.BlockSpec((BLK, HID), lambda i: (i, 0)),
            pl.BlockSpec((BLK, HID), lambda i: (i, 0)),
            pl.BlockSpec((BLK, HID), lambda i: (i, 0)),
            pl.BlockSpec((8, 2 * HID), lambda i: (0, 0)),
        ],
        out_shape=[
            jax.ShapeDtypeStruct((NPAD, HID), f32),
            jax.ShapeDtypeStruct((NPAD, HID), f32),
            jax.ShapeDtypeStruct((NPAD, HID), f32),
            jax.ShapeDtypeStruct((8, 2 * HID), f32),
        ],
    )(xp, W1, A1)


def _elu(x):
    return jnp.where(x > 0, x, jnp.exp(jnp.minimum(x, 0.0)) - 1.0)


def _dense2_body(n0_ref, n1_ref, d0_ref, d1_ref, b1_ref, w_ref, a_ref,
                 h_ref, as_ref, ad_ref, mx_ref):
    i = pl.program_id(0)
    num = n0_ref[...] + n1_ref[...]
    den = d0_ref[...] + d1_ref[...]
    g = _elu(num / (den + 1e-16) + b1_ref[...])
    h = jnp.dot(g, w_ref[...], preferred_element_type=f32)
    h_ref[...] = h
    a = jnp.dot(h, a_ref[...], preferred_element_type=f32)
    as_ref[...] = a[:, :HID]
    ad_ref[...] = a[:, HID:]
    bmax = jnp.broadcast_to(jnp.max(a, axis=0, keepdims=True), (8, 2 * HID))

    @pl.when(i == 0)
    def _():
        mx_ref[...] = bmax

    @pl.when(i > 0)
    def _():
        mx_ref[...] = jnp.maximum(mx_ref[...], bmax)


def _dense2(n0, n1, d0, d1, b1r, W2, A2):
    grid = NPAD // BLK
    return pl.pallas_call(
        _dense2_body,
        grid=(grid,),
        in_specs=[
            pl.BlockSpec((BLK, HID), lambda i: (i, 0)),
            pl.BlockSpec((BLK, HID), lambda i: (i, 0)),
            pl.BlockSpec((BLK, HID), lambda i: (i, 0)),
            pl.BlockSpec((BLK, HID), lambda i: (i, 0)),
            pl.BlockSpec((1, HID), lambda i: (0, 0)),
            pl.BlockSpec((HID, HID), lambda i: (0, 0)),
            pl.BlockSpec((HID, 2 * HID), lambda i: (0, 0)),
        ],
        out_specs=[
            pl.BlockSpec((BLK, HID), lambda i: (i, 0)),
            pl.BlockSpec((BLK, HID), lambda i: (i, 0)),
            pl.BlockSpec((BLK, HID), lambda i: (i, 0)),
            pl.BlockSpec((8, 2 * HID), lambda i: (0, 0)),
        ],
        out_shape=[
            jax.ShapeDtypeStruct((NPAD, HID), f32),
            jax.ShapeDtypeStruct((NPAD, HID), f32),
            jax.ShapeDtypeStruct((NPAD, HID), f32),
            jax.ShapeDtypeStruct((8, 2 * HID), f32),
        ],
    )(n0, n1, d0, d1, b1r, W2, A2)


def _final_body(n0_ref, n1_ref, d0_ref, d1_ref, b2_ref, o_ref):
    num = n0_ref[...] + n1_ref[...]
    den = d0_ref[...] + d1_ref[...]
    o_ref[...] = num / (den + 1e-16) + b2_ref[...]


def _final(n0, n1, d0, d1, b2r):
    grid = NPAD // BLK
    return pl.pallas_call(
        _final_body,
        grid=(grid,),
        in_specs=[
            pl.BlockSpec((BLK, HID), lambda i: (i, 0)),
            pl.BlockSpec((BLK, HID), lambda i: (i, 0)),
            pl.BlockSpec((BLK, HID), lambda i: (i, 0)),
            pl.BlockSpec((BLK, HID), lambda i: (i, 0)),
            pl.BlockSpec((1, HID), lambda i: (0, 0)),
        ],
        out_specs=pl.BlockSpec((BLK, HID), lambda i: (i, 0)),
        out_shape=jax.ShapeDtypeStruct((N, HID), f32),
    )(n0, n1, d0, d1, b2r)


def _edge_body(src_hbm, dst_hbm, h_hbm, as_hbm, ad_hbm, m_hbm,
               num_hbm, den_hbm,
               srcb, dstb, mb,
               asv0, adv0, hg0, hw0, pv0,
               asv1, adv1, hg1, hw1, pv1,
               z64, num_sh, den_sh,
               gsem0, gsem1, ssem0, ssem1):
    c = lax.axis_index("c")
    s = lax.axis_index("s")
    wid = c * 16 + s
    row0 = s * ROWS_PT

    zero = jnp.zeros((16,), f32)

    def zrow(i, _):
        for j in range(4):
            z64[i, pl.ds(16 * j, 16)] = zero
        return 0

    lax.fori_loop(0, 64, zrow, 0)
    for r in range(ROWS_PT // 64):
        pltpu.sync_copy(z64, num_sh.at[pl.ds(row0 + 64 * r, 64)])
        pltpu.sync_copy(z64, den_sh.at[pl.ds(row0 + 64 * r, 64)])
    plsc.subcore_barrier()

    pltpu.sync_copy(m_hbm, mb)
    M = [mb[pl.ds(16 * j, 16)] for j in range(4)]

    slots = [
        (asv0, adv0, hg0, hw0, pv0, gsem0, ssem0),
        (asv1, adv1, hg1, hw1, pv1, gsem1, ssem1),
    ]

    def issue_gathers(ch, b):
        asb, adb, hg, _, _, gsem, _ = slots[b]
        pltpu.async_copy(as_hbm.at[srcb.at[ch]], asb, gsem)
        pltpu.async_copy(ad_hbm.at[dstb.at[ch]], adb, gsem)
        pltpu.async_copy(h_hbm.at[srcb.at[ch]], hg, gsem)

    def wait_gathers(b):
        asb, adb, hg, _, _, gsem, _ = slots[b]
        pltpu.make_async_copy(as_hbm.at[srcb.at[0]], asb, gsem).wait()
        pltpu.make_async_copy(ad_hbm.at[dstb.at[0]], adb, gsem).wait()
        pltpu.make_async_copy(h_hbm.at[srcb.at[0]], hg, gsem).wait()

    def issue_scatters(ch, b):
        _, _, _, hw, pb, _, ssem = slots[b]
        pltpu.async_copy(hw, num_sh.at[dstb.at[ch]], ssem, add=True)
        pltpu.async_copy(pb, den_sh.at[dstb.at[ch]], ssem, add=True)

    def wait_scatters(b):
        _, _, _, hw, pb, _, ssem = slots[b]
        pltpu.make_async_copy(hw, num_sh.at[dstb.at[0]], ssem).wait()
        pltpu.make_async_copy(pb, den_sh.at[dstb.at[0]], ssem).wait()

    def compute(b):
        asb, adb, hg, hw, pb, _, _ = slots[b]

        def edge(i, _):
            for j in range(4):
                sl = pl.ds(16 * j, 16)
                u = asb[i, sl] + adb[i, sl]
                e = jnp.where(u >= 0, u, 0.2 * u)
                p = jnp.exp(e - M[j])
                pb[i, sl] = p
                hw[i, sl] = hg[i, sl] * p
            return 0

        lax.fori_loop(0, CH, edge, 0)

    def block(blk, _):
        pltpu.sync_copy(src_hbm.at[wid, pl.ds(blk * IB, IB)], srcb)
        pltpu.sync_copy(dst_hbm.at[wid, pl.ds(blk * IB, IB)], dstb)
        issue_gathers(0, 0)

        def pair(k, _):
            for b in (0, 1):
                ch = 2 * k + b
                wait_gathers(b)
                if b == 0:
                    issue_gathers(ch + 1, 1)
                else:
                    @pl.when(k < IB // 2 - 1)
                    def _():
                        issue_gathers(ch + 1, 0)

                @pl.when(k >= 1)
                def _():
                    wait_scatters(b)
                compute(b)
                issue_scatters(ch, b)
            return 0

        lax.fori_loop(0, IB // 2, pair, 0)
        wait_scatters(0)
        wait_scatters(1)
        return 0

    lax.fori_loop(0, NBLK, block, 0)
    plsc.subcore_barrier()

    pltpu.sync_copy(num_sh.at[pl.ds(row0, ROWS_PT)],
                    num_hbm.at[c, pl.ds(row0, ROWS_PT)])
    pltpu.sync_copy(den_sh.at[pl.ds(row0, ROWS_PT)],
                    den_hbm.at[c, pl.ds(row0, ROWS_PT)])


_edge = pl.kernel(
    _edge_body,
    out_type=(
        jax.ShapeDtypeStruct((2, NPAD, HID), f32),
        jax.ShapeDtypeStruct((2, NPAD, HID), f32),
    ),
    mesh=plsc.VectorSubcoreMesh(core_axis_name="c", subcore_axis_name="s",
                                num_cores=2, num_subcores=16),
    scratch_types=[
        pltpu.VMEM((IB, CH), i32),
        pltpu.VMEM((IB, CH), i32),
        pltpu.VMEM((HID,), f32),
        pltpu.VMEM((CH, HID), f32),
        pltpu.VMEM((CH, HID), f32),
        pltpu.VMEM((CH, HID), f32),
        pltpu.VMEM((CH, HID), f32),
        pltpu.VMEM((CH, HID), f32),
        pltpu.VMEM((CH, HID), f32),
        pltpu.VMEM((CH, HID), f32),
        pltpu.VMEM((CH, HID), f32),
        pltpu.VMEM((CH, HID), f32),
        pltpu.VMEM((CH, HID), f32),
        pltpu.VMEM((64, HID), f32),
        pltpu.VMEM_SHARED((NPAD, HID), f32),
        pltpu.VMEM_SHARED((NPAD, HID), f32),
        pltpu.SemaphoreType.DMA,
        pltpu.SemaphoreType.DMA,
        pltpu.SemaphoreType.DMA,
        pltpu.SemaphoreType.DMA,
    ],
    compiler_params=pltpu.CompilerParams(use_tc_tiling_on_sc=False),
)


def _lrelu(x):
    return jnp.where(x >= 0, x, 0.2 * x)


def kernel(x, edge_index, W1, a1s, a1d, b1, W2, a2s, a2d, b2):
    loops = jnp.arange(N, dtype=i32)
    src = jnp.concatenate([
        edge_index[0].astype(i32), loops,
        jnp.zeros((E_PAD - E_TOT,), i32)])
    dst = jnp.concatenate([
        edge_index[1].astype(i32), loops,
        jnp.full((E_PAD - E_TOT,), N, i32)])
    srcg = src.reshape(NW, NCH, CH)
    dstg = dst.reshape(NW, NCH, CH)

    xp = jnp.pad(x, ((0, NPAD - N), (0, 0)))

    eye8 = jnp.eye(8, dtype=f32)
    ones8 = jnp.ones((1, 1, 1, 8), f32)
    A1s = (a1s[0][:, :, None, None] * eye8[:, None, :, None] * ones8
           ).reshape(HID, HID)
    A1d = (a1d[0][:, :, None, None] * eye8[:, None, :, None] * ones8
           ).reshape(HID, HID)
    A1 = jnp.concatenate([A1s, A1d], axis=1)
    A2s = a2s[0, 0][:, None] * jnp.ones((1, HID), f32)
    A2d = a2d[0, 0][:, None] * jnp.ones((1, HID), f32)
    A2 = jnp.concatenate([A2s, A2d], axis=1)
    b1r = b1.reshape(1, HID)
    b2r = b2.reshape(1, HID)

    h1, as1, ad1, mx1 = _dense1(xp, W1, A1)
    m1 = _lrelu(mx1[0, :HID] + mx1[0, HID:])
    num1, den1 = _edge(srcg, dstg, h1, as1, ad1, m1)

    h2, as2, ad2, mx2 = _dense2(num1[0], num1[1], den1[0], den1[1],
                                b1r, W2, A2)
    m2 = _lrelu(mx2[0, :HID] + mx2[0, HID:])
    num2, den2 = _edge(srcg, dstg, h2, as2, ad2, m2)

    return _final(num2[0], num2[1], den2[0], den2[1], b2r)


# speedup vs baseline: 1.0629x; 1.0141x over previous
import jax
import jax.numpy as jnp
from jax import lax
from jax.experimental import pallas as pl
from jax.experimental.pallas import tpu as pltpu
from jax.experimental.pallas import tpu_sc as plsc

N = 10000
NPAD = 10240
D_IN = 128
HID = 64
E = 320000
E_TOT = E + N
NW = 32
CH = 64
IB = 18
NBLK = 9
NCH = IB * NBLK
EPW = NCH * CH
E_PAD = EPW * NW
ROWS_PT = NPAD // 16
BLK = 1024

f32 = jnp.float32
i32 = jnp.int32


def _dense1_body(x_ref, w_ref, a_ref, h_ref, as_ref, ad_ref, mx_ref):
    i = pl.program_id(0)
    h = jnp.dot(x_ref[...], w_ref[...], preferred_element_type=f32)
    h_ref[...] = h
    a = jnp.dot(h, a_ref[...], preferred_element_type=f32)
    as_ref[...] = a[:, :HID]
    ad_ref[...] = a[:, HID:]
    bmax = jnp.broadcast_to(jnp.max(a, axis=0, keepdims=True), (8, 2 * HID))

    @pl.when(i == 0)
    def _():
        mx_ref[...] = bmax

    @pl.when(i > 0)
    def _():
        mx_ref[...] = jnp.maximum(mx_ref[...], bmax)


def _dense1(xp, W1, A1):
    grid = NPAD // BLK
    return pl.pallas_call(
        _dense1_body,
        grid=(grid,),
        in_specs=[
            pl.BlockSpec((BLK, D_IN), lambda i: (i, 0)),
            pl.BlockSpec((D_IN, HID), lambda i: (0, 0)),
            pl.BlockSpec((HID, 2 * HID), lambda i: (0, 0)),
        ],
        out_specs=[
            pl.BlockSpec((BLK, HID), lambda i: (i, 0)),
            pl.BlockSpec((BLK, HID), lambda i: (i, 0)),
            pl.BlockSpec((BLK, HID), lambda i: (i, 0)),
            pl.BlockSpec((8, 2 * HID), lambda i: (0, 0)),
        ],
        out_shape=[
            jax.ShapeDtypeStruct((NPAD, HID), f32),
            jax.ShapeDtypeStruct((NPAD, HID), f32),
            jax.ShapeDtypeStruct((NPAD, HID), f32),
            jax.ShapeDtypeStruct((8, 2 * HID), f32),
        ],
    )(xp, W1, A1)


def _elu(x):
    return jnp.where(x > 0, x, jnp.exp(jnp.minimum(x, 0.0)) - 1.0)


def _dense2_body(n0_ref, n1_ref, d0_ref, d1_ref, b1_ref, w_ref, a_ref,
                 h_ref, as_ref, ad_ref, mx_ref):
    i = pl.program_id(0)
    num = n0_ref[...] + n1_ref[...]
    den = d0_ref[...] + d1_ref[...]
    g = _elu(num / (den + 1e-16) + b1_ref[...])
    h = jnp.dot(g, w_ref[...], preferred_element_type=f32)
    h_ref[...] = h
    a = jnp.dot(h, a_ref[...], preferred_element_type=f32)
    as_ref[...] = a[:, :HID]
    ad_ref[...] = a[:, HID:]
    bmax = jnp.broadcast_to(jnp.max(a, axis=0, keepdims=True), (8, 2 * HID))

    @pl.when(i == 0)
    def _():
        mx_ref[...] = bmax

    @pl.when(i > 0)
    def _():
        mx_ref[...] = jnp.maximum(mx_ref[...], bmax)


def _dense2(n0, n1, d0, d1, b1r, W2, A2):
    grid = NPAD // BLK
    return pl.pallas_call(
        _dense2_body,
        grid=(grid,),
        in_specs=[
            pl.BlockSpec((BLK, HID), lambda i: (i, 0)),
            pl.BlockSpec((BLK, HID), lambda i: (i, 0)),
            pl.BlockSpec((BLK, HID), lambda i: (i, 0)),
            pl.BlockSpec((BLK, HID), lambda i: (i, 0)),
            pl.BlockSpec((1, HID), lambda i: (0, 0)),
            pl.BlockSpec((HID, HID), lambda i: (0, 0)),
            pl.BlockSpec((HID, 2 * HID), lambda i: (0, 0)),
        ],
        out_specs=[
            pl.BlockSpec((BLK, HID), lambda i: (i, 0)),
            pl.BlockSpec((BLK, HID), lambda i: (i, 0)),
            pl.BlockSpec((BLK, HID), lambda i: (i, 0)),
            pl.BlockSpec((8, 2 * HID), lambda i: (0, 0)),
        ],
        out_shape=[
            jax.ShapeDtypeStruct((NPAD, HID), f32),
            jax.ShapeDtypeStruct((NPAD, HID), f32),
            jax.ShapeDtypeStruct((NPAD, HID), f32),
            jax.ShapeDtypeStruct((8, 2 * HID), f32),
        ],
    )(n0, n1, d0, d1, b1r, W2, A2)


def _final_body(n0_ref, n1_ref, d0_ref, d1_ref, b2_ref, o_ref):
    num = n0_ref[...] + n1_ref[...]
    den = d0_ref[...] + d1_ref[...]
    o_ref[...] = num / (den + 1e-16) + b2_ref[...]


def _final(n0, n1, d0, d1, b2r):
    grid = NPAD // BLK
    return pl.pallas_call(
        _final_body,
        grid=(grid,),
        in_specs=[
            pl.BlockSpec((BLK, HID), lambda i: (i, 0)),
            pl.BlockSpec((BLK, HID), lambda i: (i, 0)),
            pl.BlockSpec((BLK, HID), lambda i: (i, 0)),
            pl.BlockSpec((BLK, HID), lambda i: (i, 0)),
            pl.BlockSpec((1, HID), lambda i: (0, 0)),
        ],
        out_specs=pl.BlockSpec((BLK, HID), lambda i: (i, 0)),
        out_shape=jax.ShapeDtypeStruct((NPAD, HID), f32),
    )(n0, n1, d0, d1, b2r)


def _edge_body(src_hbm, dst_hbm, h_hbm, as_hbm, ad_hbm, m_hbm,
               num_hbm, den_hbm,
               srcb, dstb, mb,
               asv0, adv0, hg0, hw0, pv0,
               asv1, adv1, hg1, hw1, pv1,
               z64, num_sh, den_sh,
               gsem0, gsem1, ssem0, ssem1):
    c = lax.axis_index("c")
    s = lax.axis_index("s")
    wid = c * 16 + s
    row0 = s * ROWS_PT

    zero = jnp.zeros((16,), f32)

    def zrow(i, _):
        for j in range(4):
            z64[i, pl.ds(16 * j, 16)] = zero
        return 0

    lax.fori_loop(0, 64, zrow, 0)
    for r in range(ROWS_PT // 64):
        pltpu.sync_copy(z64, num_sh.at[pl.ds(row0 + 64 * r, 64)])
        pltpu.sync_copy(z64, den_sh.at[pl.ds(row0 + 64 * r, 64)])
    plsc.subcore_barrier()

    pltpu.sync_copy(m_hbm, mb)
    M = [mb[pl.ds(16 * j, 16)] for j in range(4)]

    slots = [
        (asv0, adv0, hg0, hw0, pv0, gsem0, ssem0),
        (asv1, adv1, hg1, hw1, pv1, gsem1, ssem1),
    ]

    def issue_gathers(ch, b):
        asb, adb, hg, _, _, gsem, _ = slots[b]
        pltpu.async_copy(as_hbm.at[srcb.at[ch]], asb, gsem)
        pltpu.async_copy(ad_hbm.at[dstb.at[ch]], adb, gsem)
        pltpu.async_copy(h_hbm.at[srcb.at[ch]], hg, gsem)

    def wait_gathers(b):
        asb, adb, hg, _, _, gsem, _ = slots[b]
        pltpu.make_async_copy(as_hbm.at[srcb.at[0]], asb, gsem).wait()
        pltpu.make_async_copy(ad_hbm.at[dstb.at[0]], adb, gsem).wait()
        pltpu.make_async_copy(h_hbm.at[srcb.at[0]], hg, gsem).wait()

    def issue_scatters(ch, b):
        _, _, _, hw, pb, _, ssem = slots[b]
        pltpu.async_copy(hw, num_sh.at[dstb.at[ch]], ssem, add=True)
        pltpu.async_copy(pb, den_sh.at[dstb.at[ch]], ssem, add=True)

    def wait_scatters(b):
        _, _, _, hw, pb, _, ssem = slots[b]
        pltpu.make_async_copy(hw, num_sh.at[dstb.at[0]], ssem).wait()
        pltpu.make_async_copy(pb, den_sh.at[dstb.at[0]], ssem).wait()

    def compute(b):
        asb, adb, hg, hw, pb, _, _ = slots[b]

        def edge(i, _):
            for j in range(4):
                sl = pl.ds(16 * j, 16)
                u = asb[i, sl] + adb[i, sl]
                e = jnp.where(u >= 0, u, 0.2 * u)
                p = jnp.exp(e - M[j])
                pb[i, sl] = p
                hw[i, sl] = hg[i, sl] * p
            return 0

        lax.fori_loop(0, CH, edge, 0)

    def block(blk, _):
        pltpu.sync_copy(src_hbm.at[wid, pl.ds(blk * IB, IB)], srcb)
        pltpu.sync_copy(dst_hbm.at[wid, pl.ds(blk * IB, IB)], dstb)
        issue_gathers(0, 0)

        def pair(k, _):
            for b in (0, 1):
                ch = 2 * k + b
                wait_gathers(b)
                if b == 0:
                    issue_gathers(ch + 1, 1)
                else:
                    @pl.when(k < IB // 2 - 1)
                    def _():
                        issue_gathers(ch + 1, 0)

                @pl.when(k >= 1)
                def _():
                    wait_scatters(b)
                compute(b)
                issue_scatters(ch, b)
            return 0

        lax.fori_loop(0, IB // 2, pair, 0)
        wait_scatters(0)
        wait_scatters(1)
        return 0

    lax.fori_loop(0, NBLK, block, 0)
    plsc.subcore_barrier()

    pltpu.sync_copy(num_sh.at[pl.ds(row0, ROWS_PT)],
                    num_hbm.at[c, pl.ds(row0, ROWS_PT)])
    pltpu.sync_copy(den_sh.at[pl.ds(row0, ROWS_PT)],
                    den_hbm.at[c, pl.ds(row0, ROWS_PT)])


_edge = pl.kernel(
    _edge_body,
    out_type=(
        jax.ShapeDtypeStruct((2, NPAD, HID), f32),
        jax.ShapeDtypeStruct((2, NPAD, HID), f32),
    ),
    mesh=plsc.VectorSubcoreMesh(core_axis_name="c", subcore_axis_name="s",
                                num_cores=2, num_subcores=16),
    scratch_types=[
        pltpu.VMEM((IB, CH), i32),
        pltpu.VMEM((IB, CH), i32),
        pltpu.VMEM((HID,), f32),
        pltpu.VMEM((CH, HID), f32),
        pltpu.VMEM((CH, HID), f32),
        pltpu.VMEM((CH, HID), f32),
        pltpu.VMEM((CH, HID), f32),
        pltpu.VMEM((CH, HID), f32),
        pltpu.VMEM((CH, HID), f32),
        pltpu.VMEM((CH, HID), f32),
        pltpu.VMEM((CH, HID), f32),
        pltpu.VMEM((CH, HID), f32),
        pltpu.VMEM((CH, HID), f32),
        pltpu.VMEM((64, HID), f32),
        pltpu.VMEM_SHARED((NPAD, HID), f32),
        pltpu.VMEM_SHARED((NPAD, HID), f32),
        pltpu.SemaphoreType.DMA,
        pltpu.SemaphoreType.DMA,
        pltpu.SemaphoreType.DMA,
        pltpu.SemaphoreType.DMA,
    ],
    compiler_params=pltpu.CompilerParams(use_tc_tiling_on_sc=False),
)


def _lrelu(x):
    return jnp.where(x >= 0, x, 0.2 * x)


def kernel(x, edge_index, W1, a1s, a1d, b1, W2, a2s, a2d, b2):
    loops = jnp.arange(N, dtype=i32)
    src = jnp.concatenate([
        edge_index[0].astype(i32), loops,
        jnp.zeros((E_PAD - E_TOT,), i32)])
    dst = jnp.concatenate([
        edge_index[1].astype(i32), loops,
        jnp.full((E_PAD - E_TOT,), N, i32)])
    srcg = src.reshape(NW, NCH, CH)
    dstg = dst.reshape(NW, NCH, CH)

    xp = jnp.pad(x, ((0, NPAD - N), (0, 0)))

    eye8 = jnp.eye(8, dtype=f32)
    ones8 = jnp.ones((1, 1, 1, 8), f32)
    A1s = (a1s[0][:, :, None, None] * eye8[:, None, :, None] * ones8
           ).reshape(HID, HID)
    A1d = (a1d[0][:, :, None, None] * eye8[:, None, :, None] * ones8
           ).reshape(HID, HID)
    A1 = jnp.concatenate([A1s, A1d], axis=1)
    A2s = a2s[0, 0][:, None] * jnp.ones((1, HID), f32)
    A2d = a2d[0, 0][:, None] * jnp.ones((1, HID), f32)
    A2 = jnp.concatenate([A2s, A2d], axis=1)
    b1r = b1.reshape(1, HID)
    b2r = b2.reshape(1, HID)

    h1, as1, ad1, mx1 = _dense1(xp, W1, A1)
    m1 = _lrelu(mx1[0, :HID] + mx1[0, HID:])
    num1, den1 = _edge(srcg, dstg, h1, as1, ad1, m1)

    h2, as2, ad2, mx2 = _dense2(num1[0], num1[1], den1[0], den1[1],
                                b1r, W2, A2)
    m2 = _lrelu(mx2[0, :HID] + mx2[0, HID:])
    num2, den2 = _edge(srcg, dstg, h2, as2, ad2, m2)

    out = _final(num2[0], num2[1], den2[0], den2[1], b2r)
    return out[:N]
